# baseline (speedup 1.0000x reference)
import functools

import jax
import jax.numpy as jnp
from jax import lax
from jax.experimental import pallas as pl
from jax.experimental.pallas import tpu as pltpu
from jax.experimental.pallas import tpu_sc as plsc

EMBED = 128
BATCH = 16384
NUM_TABLES = 6

SC_ROWS = 4096
TC_ROWS = BATCH - SC_ROWS

NUM_CORES = 2
NUM_SUBCORES = 16
NUM_WORKERS = NUM_CORES * NUM_SUBCORES
ROWS_PER_WORKER = SC_ROWS // NUM_WORKERS
CHUNK = 128
NUM_CHUNKS = ROWS_PER_WORKER // CHUNK
IDS_ROW0 = TC_ROWS // CHUNK


def _sc_body(ids_hbm, tables_hbm, out_hbm, idx_v, rows_v, tab_sh, sem):
    wid = lax.axis_index("s") * NUM_CORES + lax.axis_index("c")
    base = wid * ROWS_PER_WORKER
    pltpu.async_copy(tables_hbm, tab_sh, sem)
    pltpu.async_copy(
        ids_hbm.at[pl.ds(IDS_ROW0 + wid * NUM_CHUNKS, NUM_CHUNKS)], idx_v, sem
    )
    pltpu.make_async_copy(tables_hbm, tab_sh, sem).wait()
    pltpu.make_async_copy(
        ids_hbm.at[pl.ds(IDS_ROW0 + wid * NUM_CHUNKS, NUM_CHUNKS)], idx_v, sem
    ).wait()
    for c in range(NUM_CHUNKS):
        pltpu.async_copy(
            tab_sh.at[idx_v.at[c]], rows_v.at[pl.ds(c * CHUNK, CHUNK)], sem
        )
    for c in range(NUM_CHUNKS):
        pltpu.make_async_copy(
            tab_sh.at[idx_v.at[c]], rows_v.at[pl.ds(c * CHUNK, CHUNK)], sem
        ).wait()
    pltpu.async_copy(rows_v, out_hbm.at[pl.ds(base, ROWS_PER_WORKER)], sem).wait()


TC_BLK = 1024
TC_GRID = TC_ROWS // TC_BLK


def _tc_body(ids_ref, tab_ref, out_ref):
    ids = ids_ref[0, 0, :].reshape(TC_BLK, 1)
    acc = jnp.zeros((TC_BLK, EMBED), jnp.float32)
    for d in range(NUM_TABLES):
        acc = acc + jnp.where(ids == d, 1.0, 0.0) * tab_ref[d, :].reshape(1, EMBED)
    out_ref[...] = acc


@jax.jit
def _run(ids2d, tables):
    mesh = plsc.VectorSubcoreMesh(core_axis_name="c", subcore_axis_name="s")
    sc_out = pl.kernel(
        _sc_body,
        mesh=mesh,
        out_type=jax.ShapeDtypeStruct((SC_ROWS, EMBED), jnp.float32),
        scratch_types=[
            pltpu.VMEM((NUM_CHUNKS, CHUNK), jnp.int32),
            pltpu.VMEM((ROWS_PER_WORKER, EMBED), jnp.float32),
            pltpu.VMEM_SHARED((NUM_TABLES, EMBED), jnp.float32),
            pltpu.SemaphoreType.DMA,
        ],
    )(ids2d, tables)
    tc_out = pl.pallas_call(
        _tc_body,
        grid=(TC_GRID,),
        in_specs=[
            pl.BlockSpec((1, 1, TC_BLK), lambda i: (i, 0, 0)),
            pl.BlockSpec((NUM_TABLES, EMBED), lambda i: (0, 0)),
        ],
        out_specs=pl.BlockSpec((TC_BLK, EMBED), lambda i: (i, 0)),
        out_shape=jax.ShapeDtypeStruct((TC_ROWS, EMBED), jnp.float32),
    )(ids2d.reshape(TC_GRID + SC_ROWS // TC_BLK, 1, TC_BLK), tables)
    return jnp.concatenate([tc_out, sc_out], axis=0)


def kernel(dataset_ids, tables):
    ids2d = dataset_ids.astype(jnp.int32).reshape(BATCH // CHUNK, CHUNK)
    return _run(ids2d, tables)


# speedup vs baseline: 1.1198x; 1.1198x over previous
import functools

import jax
import jax.numpy as jnp
from jax import lax
from jax.experimental import pallas as pl
from jax.experimental.pallas import tpu as pltpu
from jax.experimental.pallas import tpu_sc as plsc

EMBED = 128
BATCH = 16384
NUM_TABLES = 6

SC_ROWS = 4096
TC_ROWS = BATCH - SC_ROWS

NUM_CORES = 2
NUM_SUBCORES = 16
NUM_WORKERS = NUM_CORES * NUM_SUBCORES
ROWS_PER_WORKER = SC_ROWS // NUM_WORKERS
CHUNK = 128
NUM_CHUNKS = ROWS_PER_WORKER // CHUNK
IDS_ROW0 = TC_ROWS // CHUNK


def _sc_body(ids_hbm, tables_hbm, out_hbm, idx_v, rows_v, tab_sh, sem):
    wid = lax.axis_index("s") * NUM_CORES + lax.axis_index("c")
    base = wid * ROWS_PER_WORKER
    pltpu.async_copy(tables_hbm, tab_sh, sem)
    pltpu.async_copy(
        ids_hbm.at[pl.ds(IDS_ROW0 + wid * NUM_CHUNKS, NUM_CHUNKS)], idx_v, sem
    )
    pltpu.make_async_copy(tables_hbm, tab_sh, sem).wait()
    pltpu.make_async_copy(
        ids_hbm.at[pl.ds(IDS_ROW0 + wid * NUM_CHUNKS, NUM_CHUNKS)], idx_v, sem
    ).wait()
    for c in range(NUM_CHUNKS):
        pltpu.async_copy(
            tab_sh.at[idx_v.at[c]], rows_v.at[pl.ds(c * CHUNK, CHUNK)], sem
        )
    for c in range(NUM_CHUNKS):
        pltpu.make_async_copy(
            tab_sh.at[idx_v.at[c]], rows_v.at[pl.ds(c * CHUNK, CHUNK)], sem
        ).wait()
    pltpu.async_copy(rows_v, out_hbm.at[pl.ds(base, ROWS_PER_WORKER)], sem).wait()


TC_BLK = 1024
TC_GRID = TC_ROWS // TC_BLK


def _tc_body(ids_ref, tab_ref, out_ref):
    ids = ids_ref[0, 0, :].reshape(TC_BLK, 1)
    acc = jnp.zeros((TC_BLK, EMBED), jnp.float32)
    for d in range(NUM_TABLES):
        acc = acc + jnp.where(ids == d, 1.0, 0.0) * tab_ref[d, :].reshape(1, EMBED)
    out_ref[...] = acc


@jax.jit
def _run(ids2d, tables):
    mesh = plsc.VectorSubcoreMesh(core_axis_name="c", subcore_axis_name="s")
    sc_out = pl.kernel(
        _sc_body,
        mesh=mesh,
        out_type=jax.ShapeDtypeStruct((SC_ROWS, EMBED), jnp.float32),
        scratch_types=[
            pltpu.VMEM((NUM_CHUNKS, CHUNK), jnp.int32),
            pltpu.VMEM((ROWS_PER_WORKER, EMBED), jnp.float32),
            pltpu.VMEM_SHARED((NUM_TABLES, EMBED), jnp.float32),
            pltpu.SemaphoreType.DMA,
        ],
    )(ids2d, tables)
    tc_out = pl.pallas_call(
        _tc_body,
        grid=(TC_GRID,),
        in_specs=[
            pl.BlockSpec((1, 1, TC_BLK), lambda i: (i, 0, 0)),
            pl.BlockSpec((NUM_TABLES, EMBED), lambda i: (0, 0)),
        ],
        out_specs=pl.BlockSpec((TC_BLK, EMBED), lambda i: (i, 0)),
        out_shape=jax.ShapeDtypeStruct((BATCH, EMBED), jnp.float32),
    )(ids2d.reshape(BATCH // TC_BLK, 1, TC_BLK), tables)
    return lax.dynamic_update_slice(tc_out, sc_out, (TC_ROWS, 0))


def kernel(dataset_ids, tables):
    ids2d = dataset_ids.astype(jnp.int32).reshape(BATCH // CHUNK, CHUNK)
    return _run(ids2d, tables)


# speedup vs baseline: 1.2961x; 1.1575x over previous
import functools

import jax
import jax.numpy as jnp
from jax import lax
from jax.experimental import pallas as pl
from jax.experimental.pallas import tpu as pltpu
from jax.experimental.pallas import tpu_sc as plsc

EMBED = 128
BATCH = 16384
NUM_TABLES = 6
NUM_CORES = 2
NUM_SUBCORES = 16
NUM_WORKERS = NUM_CORES * NUM_SUBCORES
ROWS_PER_WORKER = BATCH // NUM_WORKERS
CHUNK = 128
NUM_CHUNKS = ROWS_PER_WORKER // CHUNK


def _sc_body(ids_hbm, tables_hbm, out_hbm, idx_v, rows_v, tab_sh, gsem, wsem):
    wid = lax.axis_index("s") * NUM_CORES + lax.axis_index("c")
    base = wid * ROWS_PER_WORKER
    pltpu.async_copy(tables_hbm, tab_sh, gsem)
    pltpu.async_copy(ids_hbm.at[pl.ds(wid * NUM_CHUNKS, NUM_CHUNKS)], idx_v, gsem)
    pltpu.make_async_copy(tables_hbm, tab_sh, gsem).wait()
    pltpu.make_async_copy(
        ids_hbm.at[pl.ds(wid * NUM_CHUNKS, NUM_CHUNKS)], idx_v, gsem
    ).wait()
    for c in range(NUM_CHUNKS):
        pltpu.async_copy(
            tab_sh.at[idx_v.at[c]], rows_v.at[pl.ds(c * CHUNK, CHUNK)], gsem
        )
    for c in range(NUM_CHUNKS):
        pltpu.make_async_copy(
            tab_sh.at[idx_v.at[c]], rows_v.at[pl.ds(c * CHUNK, CHUNK)], gsem
        ).wait()
        pltpu.async_copy(
            rows_v.at[pl.ds(c * CHUNK, CHUNK)],
            out_hbm.at[pl.ds(base + c * CHUNK, CHUNK)],
            wsem,
        )
    for c in range(NUM_CHUNKS):
        pltpu.make_async_copy(
            rows_v.at[pl.ds(c * CHUNK, CHUNK)],
            out_hbm.at[pl.ds(base + c * CHUNK, CHUNK)],
            wsem,
        ).wait()


@jax.jit
def _run(ids2d, tables):
    mesh = plsc.VectorSubcoreMesh(core_axis_name="c", subcore_axis_name="s")
    return pl.kernel(
        _sc_body,
        mesh=mesh,
        out_type=jax.ShapeDtypeStruct((BATCH, EMBED), jnp.float32),
        scratch_types=[
            pltpu.VMEM((NUM_CHUNKS, CHUNK), jnp.int32),
            pltpu.VMEM((ROWS_PER_WORKER, EMBED), jnp.float32),
            pltpu.VMEM_SHARED((NUM_TABLES, EMBED), jnp.float32),
            pltpu.SemaphoreType.DMA,
            pltpu.SemaphoreType.DMA,
        ],
    )(ids2d, tables)


def kernel(dataset_ids, tables):
    ids2d = dataset_ids.astype(jnp.int32).reshape(BATCH // CHUNK, CHUNK)
    return _run(ids2d, tables)
